# baseline (speedup 1.0000x reference)
import functools

import jax
import jax.numpy as jnp
from jax import lax
from jax.experimental import pallas as pl
from jax.experimental.pallas import tpu as pltpu
from jax.experimental.pallas import tpu_sc as plsc

N, FEAT, HID, OUT = 10000, 256, 256, 128


BM = 200


def _gcn_body(adj_ref, x_ref, w1_ref, b1_ref, w2_ref, b2_ref,
              e_ref, s1_ref, s2_ref):
    ph = pl.program_id(0)
    i = pl.program_id(1)

    @pl.when((ph == 0) & (i == 0))
    def _():
        s1_ref[...] = jnp.dot(x_ref[...], w1_ref[...],
                              preferred_element_type=jnp.float32)

    @pl.when(ph == 0)
    def _():
        h = jnp.dot(adj_ref[...], s1_ref[...],
                    preferred_element_type=jnp.float32)
        h = jnp.maximum(h + b1_ref[...], 0.0)
        s2_ref[pl.ds(i * BM, BM), :] = jnp.dot(
            h, w2_ref[...], preferred_element_type=jnp.float32)

    @pl.when(ph == 1)
    def _():
        e_ref[...] = jnp.dot(adj_ref[...], s2_ref[...],
                             preferred_element_type=jnp.float32) + b2_ref[...]


def _gcn_embeds(x, adj, W1, b1, W2, b2):
    e = pl.pallas_call(
        _gcn_body,
        grid=(2, N // BM),
        in_specs=[
            pl.BlockSpec((BM, N), lambda p, i: (i, 0)),
            pl.BlockSpec((N, FEAT), lambda p, i: (0, 0)),
            pl.BlockSpec((FEAT, HID), lambda p, i: (0, 0)),
            pl.BlockSpec((1, HID), lambda p, i: (0, 0)),
            pl.BlockSpec((HID, OUT), lambda p, i: (0, 0)),
            pl.BlockSpec((1, OUT), lambda p, i: (0, 0)),
        ],
        out_specs=pl.BlockSpec((BM, OUT), lambda p, i: (p * i, 0)),
        out_shape=jax.ShapeDtypeStruct((N, OUT), jnp.float32),
        scratch_shapes=[
            pltpu.VMEM((N, HID), jnp.float32),
            pltpu.VMEM((N, OUT), jnp.float32),
        ],
    )(adj, x, W1, b1.reshape(1, HID), W2, b2.reshape(1, OUT))
    return e


NW = 32


def _lane_perm(x, idx):
    dn = lax.GatherDimensionNumbers(
        offset_dims=(), collapsed_slice_dims=(0,), start_index_map=(0,))
    return lax.gather(x, idx[:, None], dn, (1,),
                      mode=lax.GatherScatterMode.PROMISE_IN_BOUNDS)
CHUNK = 112
GRP = CHUNK // 16


NBUF = 2


def _score_body(e_hbm, dm_hbm, isrc_hbm, idst_hbm, out_hbm,
                isrc_v, idst_v, rs_v, rd_v, out_v, dm_v, sem0, sem1):
    wid = lax.axis_index("s") * 2 + lax.axis_index("c")
    n_chunks = isrc_hbm.shape[1]
    lane = lax.broadcasted_iota(jnp.int32, (16,), 0)
    sems = (sem0, sem1)

    pltpu.sync_copy(isrc_hbm.at[wid], isrc_v)
    pltpu.sync_copy(idst_hbm.at[wid], idst_v)
    pltpu.sync_copy(dm_hbm, dm_v)
    dms = [dm_v[pl.ds(v * 16, 16)] for v in range(OUT // 16)]

    def fire(j, b):
        cps = pltpu.async_copy(e_hbm.at[isrc_v.at[j]], rs_v.at[b], sems[b])
        cpd = pltpu.async_copy(e_hbm.at[idst_v.at[j]], rd_v.at[b], sems[b])
        return cps, cpd

    def drain(b):
        pltpu.make_async_copy(e_hbm.at[isrc_v.at[0]], rs_v.at[b],
                              sems[b]).wait()
        pltpu.make_async_copy(e_hbm.at[idst_v.at[0]], rd_v.at[b],
                              sems[b]).wait()

    def compute(j, b):
        def group_body(g, carry2):
            out_vec = jnp.zeros((16,), jnp.float32)
            for t in range(16):
                c = g * 16 + t
                acc = (rs_v[b, c, pl.ds(0, 16)] * dms[0] *
                       rd_v[b, c, pl.ds(0, 16)])
                for v in range(1, OUT // 16):
                    acc = acc + (rs_v[b, c, pl.ds(v * 16, 16)] * dms[v] *
                                 rd_v[b, c, pl.ds(v * 16, 16)])
                for sh in (8, 4, 2, 1):
                    acc = acc + _lane_perm(acc, lane ^ sh)
                out_vec = jnp.where(lane == t, acc, out_vec)
            out_v[j, pl.ds(g * 16, 16)] = out_vec
            return carry2

        lax.fori_loop(0, GRP, group_body, 0)

    for b in range(NBUF):
        fire(b, b)

    def pipe_body(jj, carry):
        j0 = NBUF * jj
        for u in range(NBUF):
            j = j0 + u
            drain(u)
            compute(j, u)

            @pl.when(j + NBUF < n_chunks)
            def _():
                fire(j + NBUF, u)

        return carry

    lax.fori_loop(0, n_chunks // NBUF, pipe_body, 0)
    pltpu.sync_copy(out_v, out_hbm.at[wid])


def _score(e, dm, isrc, idst, n_chunks):
    mesh = plsc.VectorSubcoreMesh(core_axis_name="c", subcore_axis_name="s")
    f = functools.partial(
        pl.kernel,
        mesh=mesh,
        out_type=jax.ShapeDtypeStruct((NW, n_chunks, CHUNK), jnp.float32),
        scratch_types=[
            pltpu.VMEM((n_chunks, CHUNK), jnp.int32),
            pltpu.VMEM((n_chunks, CHUNK), jnp.int32),
            pltpu.VMEM((NBUF, CHUNK, OUT), jnp.float32),
            pltpu.VMEM((NBUF, CHUNK, OUT), jnp.float32),
            pltpu.VMEM((n_chunks, CHUNK), jnp.float32),
            pltpu.VMEM((OUT,), jnp.float32),
            pltpu.SemaphoreType.DMA,
            pltpu.SemaphoreType.DMA,
        ],
    )(_score_body)
    return f(e, dm, isrc, idst)


def kernel(x, adj, to_pred, W1, b1, W2, b2, distmult):
    p = to_pred.shape[0]
    per_w = ((p + NW * CHUNK - 1) // (NW * CHUNK)) * CHUNK
    n_chunks = per_w // CHUNK
    if n_chunks % NBUF:
        n_chunks += NBUF - n_chunks % NBUF
        per_w = n_chunks * CHUNK
    p_pad = NW * per_w
    pad_idx = jnp.arange(p_pad - p, dtype=jnp.int32) % N
    isrc = jnp.concatenate([to_pred[:, 0], pad_idx]).reshape(
        NW, n_chunks, CHUNK)
    idst = jnp.concatenate([to_pred[:, 1], pad_idx]).reshape(
        NW, n_chunks, CHUNK)

    e = _gcn_embeds(x, adj, W1, b1, W2, b2)
    dot = _score(e, distmult, isrc, idst, n_chunks)
    return dot.reshape(p_pad)[:p]


# speedup vs baseline: 1.0528x; 1.0528x over previous
import functools

import jax
import jax.numpy as jnp
from jax import lax
from jax.experimental import pallas as pl
from jax.experimental.pallas import tpu as pltpu
from jax.experimental.pallas import tpu_sc as plsc

N, FEAT, HID, OUT = 10000, 256, 256, 128


BM = 400


def _gcn_body(adj_ref, x_ref, w1_ref, b1_ref, w2_ref, b2_ref, dm_ref,
              e_ref, a_ref, s1_ref, s2_ref):
    ph = pl.program_id(0)
    i = pl.program_id(1)

    @pl.when((ph == 0) & (i == 0))
    def _():
        s1_ref[...] = jnp.dot(x_ref[...], w1_ref[...],
                              preferred_element_type=jnp.float32)

    @pl.when(ph == 0)
    def _():
        h = jnp.dot(adj_ref[...], s1_ref[...],
                    preferred_element_type=jnp.float32)
        h = jnp.maximum(h + b1_ref[...], 0.0)
        s2_ref[pl.ds(i * BM, BM), :] = jnp.dot(
            h, w2_ref[...], preferred_element_type=jnp.float32)

    @pl.when(ph == 1)
    def _():
        e = jnp.dot(adj_ref[...], s2_ref[...],
                    preferred_element_type=jnp.float32) + b2_ref[...]
        e_ref[...] = e
        a_ref[...] = e * dm_ref[...]


def _gcn_embeds(x, adj, W1, b1, W2, b2, distmult):
    e, a = pl.pallas_call(
        _gcn_body,
        grid=(2, N // BM),
        in_specs=[
            pl.BlockSpec((BM, N), lambda p, i: (i, 0)),
            pl.BlockSpec((N, FEAT), lambda p, i: (0, 0)),
            pl.BlockSpec((FEAT, HID), lambda p, i: (0, 0)),
            pl.BlockSpec((1, HID), lambda p, i: (0, 0)),
            pl.BlockSpec((HID, OUT), lambda p, i: (0, 0)),
            pl.BlockSpec((1, OUT), lambda p, i: (0, 0)),
            pl.BlockSpec((1, OUT), lambda p, i: (0, 0)),
        ],
        out_specs=[
            pl.BlockSpec((BM, OUT), lambda p, i: (p * i, 0)),
            pl.BlockSpec((BM, OUT), lambda p, i: (p * i, 0)),
        ],
        out_shape=[
            jax.ShapeDtypeStruct((N, OUT), jnp.float32),
            jax.ShapeDtypeStruct((N, OUT), jnp.float32),
        ],
        scratch_shapes=[
            pltpu.VMEM((N, HID), jnp.float32),
            pltpu.VMEM((N, OUT), jnp.float32),
        ],
    )(adj, x, W1, b1.reshape(1, HID), W2, b2.reshape(1, OUT),
      distmult.reshape(1, OUT))
    return e, a


NW = 32


def _lane_perm(x, idx):
    dn = lax.GatherDimensionNumbers(
        offset_dims=(), collapsed_slice_dims=(0,), start_index_map=(0,))
    return lax.gather(x, idx[:, None], dn, (1,),
                      mode=lax.GatherScatterMode.PROMISE_IN_BOUNDS)
CHUNK = 112
GRP = CHUNK // 16


NBUF = 2


def _score_body(a_hbm, e_hbm, isrc_hbm, idst_hbm, out_hbm,
                isrc_v, idst_v, rs_v, rd_v, out_v, sem0, sem1):
    wid = lax.axis_index("s") * 2 + lax.axis_index("c")
    n_chunks = isrc_hbm.shape[1]
    lane = lax.broadcasted_iota(jnp.int32, (16,), 0)
    sems = (sem0, sem1)

    pltpu.sync_copy(isrc_hbm.at[wid], isrc_v)
    pltpu.sync_copy(idst_hbm.at[wid], idst_v)

    def fire(j, b):
        cps = pltpu.async_copy(a_hbm.at[isrc_v.at[j]], rs_v.at[b], sems[b])
        cpd = pltpu.async_copy(e_hbm.at[idst_v.at[j]], rd_v.at[b], sems[b])
        return cps, cpd

    def drain(b):
        pltpu.make_async_copy(a_hbm.at[isrc_v.at[0]], rs_v.at[b],
                              sems[b]).wait()
        pltpu.make_async_copy(e_hbm.at[idst_v.at[0]], rd_v.at[b],
                              sems[b]).wait()

    def compute(j, b):
        def group_body(g, carry2):
            out_vec = jnp.zeros((16,), jnp.float32)
            for t in range(16):
                c = g * 16 + t
                acc = rs_v[b, c, pl.ds(0, 16)] * rd_v[b, c, pl.ds(0, 16)]
                for v in range(1, OUT // 16):
                    acc = acc + (rs_v[b, c, pl.ds(v * 16, 16)] *
                                 rd_v[b, c, pl.ds(v * 16, 16)])
                for sh in (8, 4, 2, 1):
                    acc = acc + _lane_perm(acc, lane ^ sh)
                out_vec = jnp.where(lane == t, acc, out_vec)
            out_v[j, pl.ds(g * 16, 16)] = out_vec
            return carry2

        lax.fori_loop(0, GRP, group_body, 0)

    for b in range(NBUF):
        fire(b, b)

    def pipe_body(jj, carry):
        j0 = NBUF * jj
        for u in range(NBUF):
            j = j0 + u
            drain(u)
            compute(j, u)

            @pl.when(j + NBUF < n_chunks)
            def _():
                fire(j + NBUF, u)

        return carry

    lax.fori_loop(0, n_chunks // NBUF, pipe_body, 0)
    pltpu.sync_copy(out_v, out_hbm.at[wid])


def _score(a, e, isrc, idst, n_chunks):
    mesh = plsc.VectorSubcoreMesh(core_axis_name="c", subcore_axis_name="s")
    f = functools.partial(
        pl.kernel,
        mesh=mesh,
        out_type=jax.ShapeDtypeStruct((NW, n_chunks, CHUNK), jnp.float32),
        scratch_types=[
            pltpu.VMEM((n_chunks, CHUNK), jnp.int32),
            pltpu.VMEM((n_chunks, CHUNK), jnp.int32),
            pltpu.VMEM((NBUF, CHUNK, OUT), jnp.float32),
            pltpu.VMEM((NBUF, CHUNK, OUT), jnp.float32),
            pltpu.VMEM((n_chunks, CHUNK), jnp.float32),
            pltpu.SemaphoreType.DMA,
            pltpu.SemaphoreType.DMA,
        ],
    )(_score_body)
    return f(a, e, isrc, idst)


def kernel(x, adj, to_pred, W1, b1, W2, b2, distmult):
    p = to_pred.shape[0]
    per_w = ((p + NW * CHUNK - 1) // (NW * CHUNK)) * CHUNK
    n_chunks = per_w // CHUNK
    if n_chunks % NBUF:
        n_chunks += NBUF - n_chunks % NBUF
        per_w = n_chunks * CHUNK
    p_pad = NW * per_w
    pad_idx = jnp.arange(p_pad - p, dtype=jnp.int32) % N
    isrc = jnp.concatenate([to_pred[:, 0], pad_idx]).reshape(
        NW, n_chunks, CHUNK)
    idst = jnp.concatenate([to_pred[:, 1], pad_idx]).reshape(
        NW, n_chunks, CHUNK)

    e, a = _gcn_embeds(x, adj, W1, b1, W2, b2, distmult)
    dot = _score(a, e, isrc, idst, n_chunks)
    return dot.reshape(p_pad)[:p]
